# baseline (speedup 1.0000x reference)
import functools

import jax
import jax.numpy as jnp
from jax import lax
from jax.experimental import pallas as pl
from jax.experimental.pallas import tpu as pltpu

N_EXP = 16
TOPK = 2
ALPHA = 0.01
HID = 2048
ROWS = 8192
BLK = 2048


def _gate_body(x_ref, w_ref, idx_ref, wgt_ref, loss_ref, pi_acc, cnt_acc):
    i = pl.program_id(0)
    nsteps = pl.num_programs(0)

    x = x_ref[...]
    w = w_ref[...]
    lt = lax.dot_general(w, x, (((1,), (1,)), ((), ())),
                         preferred_element_type=jnp.float32)

    m1 = jnp.max(lt, axis=0, keepdims=True)
    e = jnp.exp(lt - m1)
    denom = jnp.sum(e, axis=0, keepdims=True)
    rdenom = 1.0 / denom

    iota = lax.broadcasted_iota(jnp.int32, (N_EXP, BLK), 0)
    is1 = lt == m1
    idx1 = jnp.min(jnp.where(is1, iota, N_EXP), axis=0, keepdims=True)
    mask1 = iota == idx1
    neg = jnp.float32(-jnp.inf)
    l2 = jnp.where(mask1, neg, lt)
    m2 = jnp.max(l2, axis=0, keepdims=True)
    idx2 = jnp.min(jnp.where(l2 == m2, iota, N_EXP), axis=0, keepdims=True)
    mask2 = iota == idx2

    s1 = rdenom
    s2 = jnp.exp(m2 - m1) * rdenom
    d12 = s1 + s2 + 1e-20
    w1 = s1 / d12
    w2 = s2 / d12

    idx_ref[...] = jnp.concatenate([idx1, idx2], axis=0).T
    wgt_ref[...] = jnp.concatenate([w1, w2], axis=0).T

    pi_part = jnp.sum(e * rdenom, axis=1, keepdims=True)
    cnt_part = jnp.sum(mask1.astype(jnp.float32) + mask2.astype(jnp.float32),
                       axis=1, keepdims=True)

    @pl.when(i == 0)
    def _init():
        pi_acc[...] = jnp.zeros_like(pi_acc)
        cnt_acc[...] = jnp.zeros_like(cnt_acc)

    pi_acc[...] += pi_part
    cnt_acc[...] += cnt_part

    @pl.when(i == nsteps - 1)
    def _fin():
        scale = ALPHA * N_EXP / (float(ROWS) * float(ROWS * TOPK))
        loss_ref[...] = jnp.sum(pi_acc[...] * cnt_acc[...],
                                keepdims=True).reshape(1, 1) * scale


@jax.jit
def kernel(x, weight):
    xf = x.reshape(-1, HID)
    grid = (ROWS // BLK,)
    idx_t, wgt_t, loss = pl.pallas_call(
        _gate_body,
        grid=grid,
        in_specs=[
            pl.BlockSpec((BLK, HID), lambda i: (i, 0)),
            pl.BlockSpec((N_EXP, HID), lambda i: (0, 0)),
        ],
        out_specs=[
            pl.BlockSpec((BLK, TOPK), lambda i: (i, 0)),
            pl.BlockSpec((BLK, TOPK), lambda i: (i, 0)),
            pl.BlockSpec((1, 1), lambda i: (0, 0)),
        ],
        out_shape=[
            jax.ShapeDtypeStruct((ROWS, TOPK), jnp.int32),
            jax.ShapeDtypeStruct((ROWS, TOPK), jnp.float32),
            jax.ShapeDtypeStruct((1, 1), jnp.float32),
        ],
        scratch_shapes=[
            pltpu.VMEM((N_EXP, 1), jnp.float32),
            pltpu.VMEM((N_EXP, 1), jnp.float32),
        ],
        compiler_params=pltpu.CompilerParams(
            dimension_semantics=("arbitrary",),
        ),
    )(xf, weight)
    return idx_t, wgt_t, loss.reshape(())


# speedup vs baseline: 1.4176x; 1.4176x over previous
import functools

import jax
import jax.numpy as jnp
from jax import lax
from jax.experimental import pallas as pl
from jax.experimental.pallas import tpu as pltpu

N_EXP = 16
TOPK = 2
ALPHA = 0.01
HID = 2048
ROWS = 8192
BLK = 1024


def _gate_body(x_ref, w_ref, idx_ref, wgt_ref, loss_ref, pi_acc, cnt_acc):
    i = pl.program_id(0)
    nsteps = pl.num_programs(0)

    x = x_ref[...]
    w = w_ref[...]
    lt = lax.dot_general(w, x, (((1,), (1,)), ((), ())),
                         preferred_element_type=jnp.float32)

    m1 = jnp.max(lt, axis=0, keepdims=True)
    e = jnp.exp(lt - m1)
    denom = jnp.sum(e, axis=0, keepdims=True)
    rdenom = 1.0 / denom

    iota = lax.broadcasted_iota(jnp.int32, (N_EXP, BLK), 0)
    is1 = lt == m1
    idx1 = jnp.min(jnp.where(is1, iota, N_EXP), axis=0, keepdims=True)
    mask1 = iota == idx1
    neg = jnp.float32(-jnp.inf)
    l2 = jnp.where(mask1, neg, lt)
    m2 = jnp.max(l2, axis=0, keepdims=True)
    idx2 = jnp.min(jnp.where(l2 == m2, iota, N_EXP), axis=0, keepdims=True)
    mask2 = iota == idx2

    s1 = rdenom
    s2 = jnp.exp(m2 - m1) * rdenom
    d12 = s1 + s2 + 1e-20
    w1 = s1 / d12
    w2 = s2 / d12

    idx_ref[...] = jnp.concatenate([idx1, idx2], axis=0)
    wgt_ref[...] = jnp.concatenate([w1, w2], axis=0)

    pi_part = jnp.sum(e * rdenom, axis=1, keepdims=True)
    cnt_part = jnp.sum(mask1.astype(jnp.float32) + mask2.astype(jnp.float32),
                       axis=1, keepdims=True)

    @pl.when(i == 0)
    def _init():
        pi_acc[...] = jnp.zeros_like(pi_acc)
        cnt_acc[...] = jnp.zeros_like(cnt_acc)

    pi_acc[...] += pi_part
    cnt_acc[...] += cnt_part

    @pl.when(i == nsteps - 1)
    def _fin():
        scale = ALPHA * N_EXP / (float(ROWS) * float(ROWS * TOPK))
        loss_ref[...] = jnp.sum(pi_acc[...] * cnt_acc[...],
                                keepdims=True).reshape(1, 1) * scale


@jax.jit
def kernel(x, weight):
    xf = x.reshape(-1, HID)
    grid = (ROWS // BLK,)
    idx_t, wgt_t, loss = pl.pallas_call(
        _gate_body,
        grid=grid,
        in_specs=[
            pl.BlockSpec((BLK, HID), lambda i: (i, 0)),
            pl.BlockSpec((N_EXP, HID), lambda i: (0, 0)),
        ],
        out_specs=[
            pl.BlockSpec((TOPK, BLK), lambda i: (0, i)),
            pl.BlockSpec((TOPK, BLK), lambda i: (0, i)),
            pl.BlockSpec((1, 1), lambda i: (0, 0)),
        ],
        out_shape=[
            jax.ShapeDtypeStruct((TOPK, ROWS), jnp.int32),
            jax.ShapeDtypeStruct((TOPK, ROWS), jnp.float32),
            jax.ShapeDtypeStruct((1, 1), jnp.float32),
        ],
        scratch_shapes=[
            pltpu.VMEM((N_EXP, 1), jnp.float32),
            pltpu.VMEM((N_EXP, 1), jnp.float32),
        ],
        compiler_params=pltpu.CompilerParams(
            dimension_semantics=("arbitrary",),
        ),
    )(xf, weight)
    return idx_t.T, wgt_t.T, loss.reshape(())
